# baseline (speedup 1.0000x reference)
import functools

import jax
import jax.numpy as jnp
from jax import lax
from jax.experimental import pallas as pl
from jax.experimental.pallas import tpu as pltpu
from jax.experimental.pallas import tpu_sc as plsc

B = 1024
F = 26
C = 1000
D = F * C
FPAD = 32
NC = 2
NS = 16
NW = NC * NS
RPW = B // NW
L = 16


def _onehot_body(in_hbm, out_hbm, idx_v, buf0, buf1, sem0, sem1):
    wid = lax.axis_index("s") * NC + lax.axis_index("c")
    base = wid * RPW

    pltpu.sync_copy(in_hbm.at[pl.ds(base, RPW)], idx_v)

    def zinit(i, carry):
        off = pl.multiple_of(i * L, L)
        z = jnp.zeros((L,), jnp.float32)
        buf0[pl.ds(off, L)] = z
        buf1[pl.ds(off, L)] = z
        return carry

    lax.fori_loop(0, D // L, zinit, 0)

    iota = lax.iota(jnp.int32, L)
    off0 = iota * C
    off1 = jnp.minimum(iota + L, F - 1) * C
    msk1 = iota < (F - L)
    ones = jnp.ones((L,), jnp.float32)
    zval = jnp.zeros((L,), jnp.float32)

    def positions(r):
        v0 = idx_v[r, pl.ds(0, L)]
        v1 = idx_v[r, pl.ds(L, L)]
        return v0 + off0, v1 + off1

    bufs = (buf0, buf1)
    sems = (sem0, sem1)
    pending = [None, None]

    for r in range(RPW):
        b = r % 2
        buf = bufs[b]
        if pending[b] is not None:
            pending[b].wait()
            q0, q1 = positions(r - 2)
            plsc.store_scatter(buf, [q0], zval)
            plsc.store_scatter(buf, [q1], zval, mask=msk1)
        p0, p1 = positions(r)
        plsc.store_scatter(buf, [p0], ones)
        plsc.store_scatter(buf, [p1], ones, mask=msk1)
        pending[b] = pltpu.async_copy(buf, out_hbm.at[base + r], sems[b])

    pending[0].wait()
    pending[1].wait()


@jax.jit
def kernel(inputs):
    idx = inputs.astype(jnp.int32)
    idx_p = jnp.pad(idx, ((0, 0), (0, FPAD - F)))
    mesh = plsc.VectorSubcoreMesh(core_axis_name="c", subcore_axis_name="s")
    run = pl.kernel(
        _onehot_body,
        out_type=jax.ShapeDtypeStruct((B, D), jnp.float32),
        mesh=mesh,
        compiler_params=pltpu.CompilerParams(needs_layout_passes=False),
        scratch_types=[
            pltpu.VMEM((RPW, FPAD), jnp.int32),
            pltpu.VMEM((D,), jnp.float32),
            pltpu.VMEM((D,), jnp.float32),
            pltpu.SemaphoreType.DMA,
            pltpu.SemaphoreType.DMA,
        ],
    )
    return run(idx_p)


# speedup vs baseline: 1.0036x; 1.0036x over previous
import functools

import jax
import jax.numpy as jnp
from jax import lax
from jax.experimental import pallas as pl
from jax.experimental.pallas import tpu as pltpu
from jax.experimental.pallas import tpu_sc as plsc

B = 1024
F = 26
C = 1000
D = F * C
FPAD = 32
NC = 2
NS = 16
NW = NC * NS
RPW = B // NW
L = 16


def _onehot_body(in_hbm, out_hbm, idx_v, buf0, buf1, sem0, sem1):
    wid = lax.axis_index("s") * NC + lax.axis_index("c")
    base = wid * RPW

    pltpu.sync_copy(in_hbm.at[pl.ds(base, RPW)], idx_v)

    def zinit(i, carry):
        off = pl.multiple_of(i * L, L)
        z = jnp.zeros((L,), jnp.float32)
        buf0[pl.ds(off, L)] = z
        buf1[pl.ds(off, L)] = z
        return carry

    lax.fori_loop(0, D // L, zinit, 0)

    iota = lax.iota(jnp.int32, L)
    off0 = iota * C
    off1 = jnp.minimum(iota + L, F - 1) * C
    msk1 = iota < (F - L)
    ones = jnp.ones((L,), jnp.float32)
    zval = jnp.zeros((L,), jnp.float32)

    def positions(r):
        v0 = idx_v[r, pl.ds(0, L)]
        v1 = idx_v[r, pl.ds(L, L)]
        return v0 + off0, v1 + off1

    bufs = (buf0, buf1)
    sems = (sem0, sem1)
    pending = [None, None]

    for r in range(RPW):
        b = r % 2
        buf = bufs[b]
        if pending[b] is not None:
            pending[b].wait()
            q0, q1 = positions(r - 2)
            plsc.store_scatter(buf, [q0], zval)
            plsc.store_scatter(buf, [q1], zval, mask=msk1)
        p0, p1 = positions(r)
        plsc.store_scatter(buf, [p0], ones)
        plsc.store_scatter(buf, [p1], ones, mask=msk1)
        pending[b] = pltpu.async_copy(buf, out_hbm.at[base + r], sems[b])

    pending[0].wait()
    pending[1].wait()


@jax.jit
def kernel(inputs):
    idx = inputs.astype(jnp.int32)
    idx_p = jnp.pad(idx, ((0, 0), (0, FPAD - F)))
    mesh = plsc.VectorSubcoreMesh(core_axis_name="c", subcore_axis_name="s")
    run = pl.kernel(
        _onehot_body,
        out_type=jax.ShapeDtypeStruct((B, D), jnp.float32),
        mesh=mesh,
        compiler_params=pltpu.CompilerParams(
            needs_layout_passes=False, use_tc_tiling_on_sc=True),
        scratch_types=[
            pltpu.VMEM((RPW, FPAD), jnp.int32),
            pltpu.VMEM((D,), jnp.float32),
            pltpu.VMEM((D,), jnp.float32),
            pltpu.SemaphoreType.DMA,
            pltpu.SemaphoreType.DMA,
        ],
    )
    return run(idx_p)


# speedup vs baseline: 2.1426x; 2.1348x over previous
import functools

import jax
import jax.numpy as jnp
from jax import lax
from jax.experimental import pallas as pl
from jax.experimental.pallas import tpu as pltpu
from jax.experimental.pallas import tpu_sc as plsc

B = 1024
F = 26
C = 1000
D = F * C
CH = 40
NCHUNK = D // CH
NC = 2
NS = 16
NW = NC * NS
MAXC = -(-NCHUNK // NW)
L = 16


def _onehot_body(in_hbm, out_hbm, idx_v, buf0, buf1, sem0, sem1):
    wid = lax.axis_index("s") * NC + lax.axis_index("c")

    pltpu.sync_copy(in_hbm, idx_v)

    zvec = jnp.zeros((L,), jnp.float32)

    def zrow(r, carry):
        def zcol(c, carry2):
            off = pl.multiple_of(c * L, L)
            buf0[r, pl.ds(off, L)] = zvec
            buf1[r, pl.ds(off, L)] = zvec
            return carry2
        lax.fori_loop(0, B // L, zcol, 0)
        return carry

    lax.fori_loop(0, CH, zrow, 0)

    iota = lax.iota(jnp.int32, L)
    ones = jnp.ones((L,), jnp.float32)

    s = (NCHUNK * wid) >> 5
    e = (NCHUNK * (wid + 1)) >> 5

    def sweep(m, buf, val):
        j = (m * 41944) >> 20
        c0 = (m - 25 * j) * CH

        def body(i, carry):
            off = pl.multiple_of(i * L, L)
            v = idx_v[j, pl.ds(off, L)]
            crel = v - c0
            mask = (crel >= 0) & (crel < CH)
            crel_c = jnp.minimum(jnp.maximum(crel, 0), CH - 1)
            rvec = iota + i * L
            plsc.store_scatter(buf, [crel_c, rvec], val, mask=mask)
            return carry

        lax.fori_loop(0, B // L, body, 0)

    def row0_of(m):
        j = (m * 41944) >> 20
        c0 = (m - 25 * j) * CH
        return j * C + c0

    bufs = (buf0, buf1)
    sems = (sem0, sem1)

    for t in range(MAXC):
        b = t % 2
        buf = bufs[b]
        sem = sems[b]
        m = s + t

        @pl.when(m < e)
        def _():
            if t >= 2:
                pltpu.make_async_copy(
                    buf, out_hbm.at[pl.ds(row0_of(m), CH)], sem).wait()
                sweep(m - 2, buf, zvec)
            sweep(m, buf, ones)
            pltpu.async_copy(buf, out_hbm.at[pl.ds(row0_of(m), CH)], sem)

    pltpu.make_async_copy(buf0, out_hbm.at[pl.ds(0, CH)], sem0).wait()
    pltpu.make_async_copy(buf1, out_hbm.at[pl.ds(0, CH)], sem1).wait()


@jax.jit
def kernel(inputs):
    idx_t = inputs.astype(jnp.int32).T
    mesh = plsc.VectorSubcoreMesh(core_axis_name="c", subcore_axis_name="s")
    run = pl.kernel(
        _onehot_body,
        out_type=jax.ShapeDtypeStruct((D, B), jnp.float32),
        mesh=mesh,
        compiler_params=pltpu.CompilerParams(
            needs_layout_passes=False, use_tc_tiling_on_sc=True),
        scratch_types=[
            pltpu.VMEM((F, B), jnp.int32),
            pltpu.VMEM((CH, B), jnp.float32),
            pltpu.VMEM((CH, B), jnp.float32),
            pltpu.SemaphoreType.DMA,
            pltpu.SemaphoreType.DMA,
        ],
    )
    return run(idx_t).T


# speedup vs baseline: 2.3714x; 1.1068x over previous
import functools

import jax
import jax.numpy as jnp
from jax import lax
from jax.experimental import pallas as pl
from jax.experimental.pallas import tpu as pltpu
from jax.experimental.pallas import tpu_sc as plsc

B = 1024
F = 26
C = 1000
D = F * C
CH = 40
NCHUNK = D // CH
NC = 2
NS = 16
NW = NC * NS
MAXC = -(-NCHUNK // NW)
L = 16


def _onehot_body(in_hbm, out_hbm, idx_v, buf0, buf1, sem0, sem1):
    wid = lax.axis_index("s") * NC + lax.axis_index("c")

    pltpu.sync_copy(in_hbm, idx_v)

    zvec = jnp.zeros((L,), jnp.float32)
    UNR = 8

    def zero_buf(buf):
        def zrow(r, carry):
            def zcol(c, carry2):
                for k in range(UNR):
                    off = pl.multiple_of((c * UNR + k) * L, L)
                    buf[r, pl.ds(off, L)] = zvec
                return carry2
            lax.fori_loop(0, B // (L * UNR), zcol, 0)
            return carry
        lax.fori_loop(0, CH, zrow, 0)

    iota = lax.iota(jnp.int32, L)
    ones = jnp.ones((L,), jnp.float32)

    s = (NCHUNK * wid) >> 5
    e = (NCHUNK * (wid + 1)) >> 5

    def sweep(m, buf, val):
        j = (m * 41944) >> 20
        c0 = (m - 25 * j) * CH

        def body(i, carry):
            for k in range(4):
                off = pl.multiple_of((i * 4 + k) * L, L)
                v = idx_v[j, pl.ds(off, L)]
                crel = v - c0
                mask = (crel >= 0) & (crel < CH)
                crel_c = jnp.minimum(jnp.maximum(crel, 0), CH - 1)
                rvec = iota + (i * 4 + k) * L
                plsc.store_scatter(buf, [crel_c, rvec], val, mask=mask)
            return carry

        lax.fori_loop(0, B // (L * 4), body, 0)

    def row0_of(m):
        j = (m * 41944) >> 20
        c0 = (m - 25 * j) * CH
        return j * C + c0

    bufs = (buf0, buf1)
    sems = (sem0, sem1)
    nmin = NCHUNK // NW

    zero_buf(buf0)

    for t in range(MAXC):
        b = t % 2
        buf = bufs[b]
        sem = sems[b]
        m = s + t

        def do_chunk(t=t, buf=buf, sem=sem, m=m):
            if t >= 2:
                pltpu.make_async_copy(
                    buf, out_hbm.at[pl.ds(row0_of(m), CH)], sem).wait()
                sweep(m - 2, buf, zvec)
            sweep(m, buf, ones)
            pltpu.async_copy(buf, out_hbm.at[pl.ds(row0_of(m), CH)], sem)

        if t < nmin:
            do_chunk()
        else:
            pl.when(m < e)(do_chunk)
        if t == 0:
            zero_buf(buf1)

    pltpu.make_async_copy(buf0, out_hbm.at[pl.ds(0, CH)], sem0).wait()
    pltpu.make_async_copy(buf1, out_hbm.at[pl.ds(0, CH)], sem1).wait()


@jax.jit
def kernel(inputs):
    idx_t = inputs.astype(jnp.int32).T
    mesh = plsc.VectorSubcoreMesh(core_axis_name="c", subcore_axis_name="s")
    run = pl.kernel(
        _onehot_body,
        out_type=jax.ShapeDtypeStruct((D, B), jnp.float32),
        mesh=mesh,
        compiler_params=pltpu.CompilerParams(
            needs_layout_passes=False, use_tc_tiling_on_sc=True),
        scratch_types=[
            pltpu.VMEM((F, B), jnp.int32),
            pltpu.VMEM((CH, B), jnp.float32),
            pltpu.VMEM((CH, B), jnp.float32),
            pltpu.SemaphoreType.DMA,
            pltpu.SemaphoreType.DMA,
        ],
    )
    return run(idx_t).T


# speedup vs baseline: 2.5302x; 1.0670x over previous
import functools

import jax
import jax.numpy as jnp
from jax import lax
from jax.experimental import pallas as pl
from jax.experimental.pallas import tpu as pltpu
from jax.experimental.pallas import tpu_sc as plsc

B = 1024
F = 26
C = 1000
D = F * C
CH = 40
NCHUNK = D // CH
NC = 2
NS = 16
NW = NC * NS
MAXC = -(-NCHUNK // NW)
L = 16


def _onehot_body(in_hbm, out_hbm, idx_v, buf0, buf1, sem0, sem1):
    wid = lax.axis_index("s") * NC + lax.axis_index("c")

    pltpu.sync_copy(in_hbm, idx_v)

    zvec = jnp.zeros((L,), jnp.float32)
    UNR = 8

    def zero_buf(buf):
        def zrow(r, carry):
            def zcol(c, carry2):
                for k in range(UNR):
                    off = pl.multiple_of((c * UNR + k) * L, L)
                    buf[r, pl.ds(off, L)] = zvec
                return carry2
            lax.fori_loop(0, B // (L * UNR), zcol, 0)
            return carry
        lax.fori_loop(0, CH, zrow, 0)

    iota = lax.iota(jnp.int32, L)
    ones = jnp.ones((L,), jnp.float32)

    s = (NCHUNK * wid) >> 5
    e = (NCHUNK * (wid + 1)) >> 5

    def sweep(m, buf, val):
        j = (m * 41944) >> 20
        c0 = (m - 25 * j) * CH

        def body(i, carry):
            for k in range(4):
                off = pl.multiple_of((i * 4 + k) * L, L)
                v = idx_v[j, pl.ds(off, L)]
                crel = v - c0
                mask = (crel >= 0) & (crel < CH)
                crel_c = jnp.minimum(jnp.maximum(crel, 0), CH - 1)
                rvec = iota + (i * 4 + k) * L
                plsc.store_scatter(buf, [crel_c, rvec], val, mask=mask)
            return carry

        lax.fori_loop(0, B // (L * 4), body, 0)

    def row0_of(m):
        j = (m * 41944) >> 20
        c0 = (m - 25 * j) * CH
        return j * C + c0

    bufs = (buf0, buf1)
    sems = (sem0, sem1)
    nmin = NCHUNK // NW

    def start_dma(m, buf, sem):
        pltpu.async_copy(buf, out_hbm.at[pl.ds(row0_of(m), CH)], sem)

    def drain(buf, sem):
        pltpu.make_async_copy(buf, out_hbm.at[pl.ds(0, CH)], sem).wait()

    zero_buf(buf0)
    sweep(s, buf0, ones)
    start_dma(s, buf0, sem0)
    zero_buf(buf1)
    sweep(s + 1, buf1, ones)
    start_dma(s + 1, buf1, sem1)

    def pair(i, carry):
        for b in (0, 1):
            m = s + 2 * i + b
            buf, sem = bufs[b], sems[b]
            drain(buf, sem)
            sweep(m - 2, buf, zvec)
            sweep(m, buf, ones)
            start_dma(m, buf, sem)
        return carry

    lax.fori_loop(1, nmin // 2, pair, 0)

    @pl.when(s + nmin < e)
    def _():
        drain(buf0, sem0)
        sweep(s + nmin - 2, buf0, zvec)
        sweep(s + nmin, buf0, ones)
        start_dma(s + nmin, buf0, sem0)

    drain(buf0, sem0)
    drain(buf1, sem1)


@jax.jit
def kernel(inputs):
    idx_t = inputs.astype(jnp.int32).T
    mesh = plsc.VectorSubcoreMesh(core_axis_name="c", subcore_axis_name="s")
    run = pl.kernel(
        _onehot_body,
        out_type=jax.ShapeDtypeStruct((D, B), jnp.float32),
        mesh=mesh,
        compiler_params=pltpu.CompilerParams(
            needs_layout_passes=False, use_tc_tiling_on_sc=True),
        scratch_types=[
            pltpu.VMEM((F, B), jnp.int32),
            pltpu.VMEM((CH, B), jnp.float32),
            pltpu.VMEM((CH, B), jnp.float32),
            pltpu.SemaphoreType.DMA,
            pltpu.SemaphoreType.DMA,
        ],
    )
    return run(idx_t).T


# speedup vs baseline: 2.5898x; 1.0236x over previous
import functools

import jax
import jax.numpy as jnp
from jax import lax
from jax.experimental import pallas as pl
from jax.experimental.pallas import tpu as pltpu
from jax.experimental.pallas import tpu_sc as plsc

B = 1024
F = 26
C = 1000
D = F * C
CH = 40
CPF = C // CH
MAGIC = -(-(1 << 20) // CPF)
NCHUNK = D // CH
NC = 2
NS = 16
NW = NC * NS
MAXC = -(-NCHUNK // NW)
L = 16


def _onehot_body(in_hbm, out_hbm, idx_v, buf0, buf1, sem0, sem1):
    wid = lax.axis_index("s") * NC + lax.axis_index("c")

    idx_copy = pltpu.async_copy(in_hbm, idx_v, sem0)

    zvec = jnp.zeros((L,), jnp.float32)
    UNR = 8

    def zero_buf(buf):
        def zrow(r, carry):
            def zcol(c, carry2):
                for k in range(UNR):
                    off = pl.multiple_of((c * UNR + k) * L, L)
                    buf[r, pl.ds(off, L)] = zvec
                return carry2
            lax.fori_loop(0, B // (L * UNR), zcol, 0)
            return carry
        lax.fori_loop(0, CH, zrow, 0)

    iota = lax.iota(jnp.int32, L)
    ones = jnp.ones((L,), jnp.float32)

    s = (NCHUNK * wid) >> 5
    e = (NCHUNK * (wid + 1)) >> 5

    def sweep(m, buf, val):
        j = (m * MAGIC) >> 20
        c0 = (m - CPF * j) * CH

        def body(i, carry):
            for k in range(4):
                off = pl.multiple_of((i * 4 + k) * L, L)
                v = idx_v[j, pl.ds(off, L)]
                crel = v - c0
                mask = (crel >= 0) & (crel < CH)
                crel_c = jnp.minimum(jnp.maximum(crel, 0), CH - 1)
                rvec = iota + (i * 4 + k) * L
                plsc.store_scatter(buf, [crel_c, rvec], val, mask=mask)
            return carry

        lax.fori_loop(0, B // (L * 4), body, 0)

    def row0_of(m):
        j = (m * MAGIC) >> 20
        c0 = (m - CPF * j) * CH
        return j * C + c0

    bufs = (buf0, buf1)
    sems = (sem0, sem1)
    nmin = NCHUNK // NW

    def start_dma(m, buf, sem):
        pltpu.async_copy(buf, out_hbm.at[pl.ds(row0_of(m), CH)], sem)

    def drain(buf, sem):
        pltpu.make_async_copy(buf, out_hbm.at[pl.ds(0, CH)], sem).wait()

    zero_buf(buf0)
    idx_copy.wait()
    sweep(s, buf0, ones)
    start_dma(s, buf0, sem0)
    zero_buf(buf1)
    sweep(s + 1, buf1, ones)
    start_dma(s + 1, buf1, sem1)

    def pair(i, carry):
        for b in (0, 1):
            m = s + 2 * i + b
            buf, sem = bufs[b], sems[b]
            drain(buf, sem)
            sweep(m - 2, buf, zvec)
            sweep(m, buf, ones)
            start_dma(m, buf, sem)
        return carry

    lax.fori_loop(1, nmin // 2, pair, 0)

    @pl.when(s + nmin < e)
    def _():
        drain(buf0, sem0)
        sweep(s + nmin - 2, buf0, zvec)
        sweep(s + nmin, buf0, ones)
        start_dma(s + nmin, buf0, sem0)

    drain(buf0, sem0)
    drain(buf1, sem1)


@jax.jit
def kernel(inputs):
    idx_t = inputs.astype(jnp.int32).T
    mesh = plsc.VectorSubcoreMesh(core_axis_name="c", subcore_axis_name="s")
    run = pl.kernel(
        _onehot_body,
        out_type=jax.ShapeDtypeStruct((D, B), jnp.float32),
        mesh=mesh,
        compiler_params=pltpu.CompilerParams(
            needs_layout_passes=False, use_tc_tiling_on_sc=True),
        scratch_types=[
            pltpu.VMEM((F, B), jnp.int32),
            pltpu.VMEM((CH, B), jnp.float32),
            pltpu.VMEM((CH, B), jnp.float32),
            pltpu.SemaphoreType.DMA,
            pltpu.SemaphoreType.DMA,
        ],
    )
    return run(idx_t).T


# speedup vs baseline: 2.5925x; 1.0010x over previous
import functools

import jax
import jax.numpy as jnp
from jax import lax
from jax.experimental import pallas as pl
from jax.experimental.pallas import tpu as pltpu
from jax.experimental.pallas import tpu_sc as plsc

B = 1024
F = 26
C = 1000
D = F * C
CH = 40
CPF = C // CH
MAGIC = -(-(1 << 20) // CPF)
NCHUNK = D // CH
NC = 2
NS = 16
NW = NC * NS
MAXC = -(-NCHUNK // NW)
L = 16


def _onehot_body(in_hbm, out_hbm, idx_v, buf0, buf1, sem0, sem1):
    wid = lax.axis_index("s") * NC + lax.axis_index("c")

    idx_copy = pltpu.async_copy(in_hbm, idx_v, sem0)

    zvec = jnp.zeros((L,), jnp.float32)
    UNR = 8

    def zero_buf(buf):
        def zrow(r, carry):
            def zcol(c, carry2):
                for k in range(UNR):
                    off = pl.multiple_of((c * UNR + k) * L, L)
                    buf[r, pl.ds(off, L)] = zvec
                return carry2
            lax.fori_loop(0, B // (L * UNR), zcol, 0)
            return carry
        lax.fori_loop(0, CH, zrow, 0)

    iota = lax.iota(jnp.int32, L)
    ones = jnp.ones((L,), jnp.float32)

    s = (NCHUNK * wid) >> 5
    e = (NCHUNK * (wid + 1)) >> 5

    def sweep(m, buf, val):
        j = (m * MAGIC) >> 20
        c0 = (m - CPF * j) * CH

        def body(i, carry):
            for k in range(4):
                off = pl.multiple_of((i * 4 + k) * L, L)
                v = idx_v[j, pl.ds(off, L)]
                crel = v - c0
                mask = (crel >= 0) & (crel < CH)
                crel_c = jnp.minimum(jnp.maximum(crel, 0), CH - 1)
                rvec = iota + (i * 4 + k) * L
                plsc.store_scatter(buf, [crel_c, rvec], val, mask=mask)
            return carry

        lax.fori_loop(0, B // (L * 4), body, 0)

    def row0_of(m):
        j = (m * MAGIC) >> 20
        c0 = (m - CPF * j) * CH
        return j * C + c0

    bufs = (buf0, buf1)
    sems = (sem0, sem1)
    nmin = NCHUNK // NW

    def start_dma(m, buf, sem):
        pltpu.async_copy(buf, out_hbm.at[pl.ds(row0_of(m), CH)], sem)

    def drain(buf, sem):
        pltpu.make_async_copy(buf, out_hbm.at[pl.ds(0, CH)], sem).wait()

    zero_buf(buf0)
    idx_copy.wait()
    sweep(s, buf0, ones)
    start_dma(s, buf0, sem0)
    zero_buf(buf1)
    sweep(s + 1, buf1, ones)
    start_dma(s + 1, buf1, sem1)

    def pair(i, carry):
        for b in (0, 1):
            m = s + 2 * i + b
            buf, sem = bufs[b], sems[b]
            drain(buf, sem)
            sweep(m - 2, buf, zvec)
            sweep(m, buf, ones)
            start_dma(m, buf, sem)
        return carry

    lax.fori_loop(1, nmin // 2, pair, 0)

    @pl.when(s + nmin < e)
    def _():
        drain(buf0, sem0)
        sweep(s + nmin - 2, buf0, zvec)
        sweep(s + nmin, buf0, ones)
        start_dma(s + nmin, buf0, sem0)

    drain(buf0, sem0)
    drain(buf1, sem1)


@jax.jit
def kernel(inputs):
    idx_t = inputs.astype(jnp.int32).T
    mesh = plsc.VectorSubcoreMesh(core_axis_name="c", subcore_axis_name="s")
    run = pl.kernel(
        _onehot_body,
        out_type=jax.ShapeDtypeStruct((D, B), jnp.float32),
        mesh=mesh,
        compiler_params=pltpu.CompilerParams(
            needs_layout_passes=False, use_tc_tiling_on_sc=True,
            skip_device_barrier=True),
        scratch_types=[
            pltpu.VMEM((F, B), jnp.int32),
            pltpu.VMEM((CH, B), jnp.float32),
            pltpu.VMEM((CH, B), jnp.float32),
            pltpu.SemaphoreType.DMA,
            pltpu.SemaphoreType.DMA,
        ],
    )
    return run(idx_t).T
